# baseline (speedup 1.0000x reference)
import functools

import jax
import jax.numpy as jnp
from jax import lax
from jax.experimental import pallas as pl
from jax.experimental.pallas import tpu as pltpu
from jax.experimental.pallas import tpu_sc as plsc

NUM_USERS = 5000
NUM_ITEMS = 4000
NUM_INGRE = 1000
D = 128
N_LAYERS = 3
N_EDGES = 320000
N_NODES = NUM_USERS + NUM_ITEMS + NUM_INGRE

NC = 2
NS = 16
L = 16
NW = NC * NS
E_PER_W = N_EDGES // NW
EB = 80
N_BLOCKS = E_PER_W // EB
N_PAD = 10240
ROWS_PER_TILE = N_PAD // NS
ZROWS = 128


def _sc_layer_body(emb_hbm, src_hbm, dst_hbm, w_hbm, out_hbm,
                   src_v, dst_v, w_v, rows_v, zero_v, acc_sh, sem):
  c = lax.axis_index("c")
  s = lax.axis_index("s")
  wid = s * NC + c

  def _zero_row(r, _):
    for k in range(D // L):
      zero_v[r, pl.ds(k * L, L)] = jnp.zeros((L,), jnp.float32)
    return 0
  lax.fori_loop(0, ZROWS, _zero_row, 0)
  for j in range(ROWS_PER_TILE // ZROWS):
    pltpu.sync_copy(zero_v, acc_sh.at[pl.ds(s * ROWS_PER_TILE + j * ZROWS,
                                            ZROWS)])
  plsc.subcore_barrier()

  def _block(i, _):
    base = wid * E_PER_W + i * EB
    pltpu.sync_copy(src_hbm.at[pl.ds(base, EB)], src_v)
    pltpu.sync_copy(dst_hbm.at[pl.ds(base, EB)], dst_v)
    pltpu.sync_copy(w_hbm.at[pl.ds(base, EB)], w_v)
    pltpu.async_copy(emb_hbm.at[src_v], rows_v, sem).wait()

    def _scale(g, _):
      w16 = w_v[pl.ds(g * L, L)]
      for j in range(L):
        wb = w16[j]
        b = g * L + j
        for k in range(D // L):
          rows_v[b, pl.ds(k * L, L)] = rows_v[b, pl.ds(k * L, L)] * wb
      return 0
    lax.fori_loop(0, EB // L, _scale, 0)

    pltpu.sync_copy(rows_v, acc_sh.at[dst_v], add=True)
    return 0
  lax.fori_loop(0, N_BLOCKS, _block, 0)
  plsc.subcore_barrier()

  r0 = s * ROWS_PER_TILE
  @pl.when(s < NS - 1)
  def _():
    pltpu.sync_copy(acc_sh.at[pl.ds(r0, ROWS_PER_TILE)],
                    out_hbm.at[c, pl.ds(r0, ROWS_PER_TILE)])
  @pl.when(s == NS - 1)
  def _():
    pltpu.sync_copy(acc_sh.at[pl.ds(r0, N_NODES - (NS - 1) * ROWS_PER_TILE)],
                    out_hbm.at[c, pl.ds(r0, N_NODES - (NS - 1) * ROWS_PER_TILE)])


_sc_layer = functools.partial(
    pl.kernel,
    out_type=jax.ShapeDtypeStruct((NC, N_NODES, D), jnp.float32),
    mesh=plsc.VectorSubcoreMesh(core_axis_name="c", subcore_axis_name="s",
                                num_cores=NC, num_subcores=NS),
    scratch_types=[
        pltpu.VMEM((EB,), jnp.int32),
        pltpu.VMEM((EB,), jnp.int32),
        pltpu.VMEM((EB,), jnp.float32),
        pltpu.VMEM((EB, D), jnp.float32),
        pltpu.VMEM((ZROWS, D), jnp.float32),
        pltpu.VMEM_SHARED((N_PAD, D), jnp.float32),
        pltpu.SemaphoreType.DMA,
    ],
)(_sc_layer_body)


def _combine_body(last, p_ref, acc_ref, e_ref, acc_out_ref):
  e = p_ref[0] + p_ref[1]
  e_ref[...] = e
  a = acc_ref[...] + e
  if last:
    a = a * jnp.float32(1.0 / (N_LAYERS + 1))
  acc_out_ref[...] = a


def _combine(p, acc, last):
  rb = 1000
  grid = (N_NODES // rb,)
  return pl.pallas_call(
      functools.partial(_combine_body, last),
      grid=grid,
      in_specs=[
          pl.BlockSpec((NC, rb, D), lambda i: (0, i, 0)),
          pl.BlockSpec((rb, D), lambda i: (i, 0)),
      ],
      out_specs=[
          pl.BlockSpec((rb, D), lambda i: (i, 0)),
          pl.BlockSpec((rb, D), lambda i: (i, 0)),
      ],
      out_shape=[
          jax.ShapeDtypeStruct((N_NODES, D), jnp.float32),
          jax.ShapeDtypeStruct((N_NODES, D), jnp.float32),
      ],
  )(p, acc)


@jax.jit
def kernel(user_emb, item_emb, ingre_emb, edge_values, edge_index):
  all0 = jnp.concatenate([user_emb, item_emb, ingre_emb], axis=0)
  src = edge_index[0]
  dst = edge_index[1]
  emb = all0
  acc = all0
  for layer in range(N_LAYERS):
    p = _sc_layer(emb, src, dst, edge_values)
    emb, acc = _combine(p, acc, layer == N_LAYERS - 1)
  return (acc[:NUM_USERS],
          acc[NUM_USERS:NUM_USERS + NUM_ITEMS],
          acc[NUM_USERS + NUM_ITEMS:])


# speedup vs baseline: 1.3011x; 1.3011x over previous
import functools

import jax
import jax.numpy as jnp
from jax import lax
from jax.experimental import pallas as pl
from jax.experimental.pallas import tpu as pltpu
from jax.experimental.pallas import tpu_sc as plsc

NUM_USERS = 5000
NUM_ITEMS = 4000
NUM_INGRE = 1000
D = 128
N_LAYERS = 3
N_EDGES = 320000
N_NODES = NUM_USERS + NUM_ITEMS + NUM_INGRE

NC = 2
NS = 16
L = 16
NW = NC * NS
E_PER_W = N_EDGES // NW
EB = 80
N_BLOCKS = E_PER_W // EB
N_PAD = 10240
ROWS_PER_TILE = N_PAD // NS
ZROWS = 128


def _sc_layer_body(emb_hbm, src_hbm, dst_hbm, w_hbm, out_hbm,
                   src_v, dst_v, w_v, rows_v, zero_v, acc_sh, sem, sem_i):
  c = lax.axis_index("c")
  s = lax.axis_index("s")
  wid = s * NC + c

  def _zero_row(r, _):
    for k in range(D // L):
      zero_v[r, pl.ds(k * L, L)] = jnp.zeros((L,), jnp.float32)
    return 0
  lax.fori_loop(0, ZROWS, _zero_row, 0)
  for j in range(ROWS_PER_TILE // ZROWS):
    pltpu.sync_copy(zero_v, acc_sh.at[pl.ds(s * ROWS_PER_TILE + j * ZROWS,
                                            ZROWS)])
  plsc.subcore_barrier()

  def _block(i, _):
    base = wid * E_PER_W + i * EB
    pltpu.async_copy(src_hbm.at[pl.ds(base, EB)], src_v, sem_i)
    pltpu.async_copy(dst_hbm.at[pl.ds(base, EB)], dst_v, sem_i)
    pltpu.async_copy(w_hbm.at[pl.ds(base, EB)], w_v, sem_i)
    pltpu.make_async_copy(src_hbm.at[pl.ds(0, EB)], src_v, sem_i).wait()
    pltpu.make_async_copy(src_hbm.at[pl.ds(0, EB)], dst_v, sem_i).wait()
    pltpu.make_async_copy(w_hbm.at[pl.ds(0, EB)], w_v, sem_i).wait()
    pltpu.async_copy(emb_hbm.at[src_v], rows_v, sem).wait()

    def _scale(g, _):
      w16 = w_v[pl.ds(g * L, L)]
      for j in range(L):
        wb = w16[j]
        b = g * L + j
        for k in range(D // L):
          rows_v[b, pl.ds(k * L, L)] = rows_v[b, pl.ds(k * L, L)] * wb
      return 0
    lax.fori_loop(0, EB // L, _scale, 0)

    pltpu.sync_copy(rows_v, acc_sh.at[dst_v], add=True)
    return 0
  lax.fori_loop(0, N_BLOCKS, _block, 0)
  plsc.subcore_barrier()

  r0 = s * ROWS_PER_TILE
  @pl.when(s < NS - 1)
  def _():
    pltpu.sync_copy(acc_sh.at[pl.ds(r0, ROWS_PER_TILE)],
                    out_hbm.at[c, pl.ds(r0, ROWS_PER_TILE)])
  @pl.when(s == NS - 1)
  def _():
    pltpu.sync_copy(acc_sh.at[pl.ds(r0, N_NODES - (NS - 1) * ROWS_PER_TILE)],
                    out_hbm.at[c, pl.ds(r0, N_NODES - (NS - 1) * ROWS_PER_TILE)])


_sc_layer = functools.partial(
    pl.kernel,
    out_type=jax.ShapeDtypeStruct((NC, N_NODES, D), jnp.float32),
    mesh=plsc.VectorSubcoreMesh(core_axis_name="c", subcore_axis_name="s",
                                num_cores=NC, num_subcores=NS),
    scratch_types=[
        pltpu.VMEM((EB,), jnp.int32),
        pltpu.VMEM((EB,), jnp.int32),
        pltpu.VMEM((EB,), jnp.float32),
        pltpu.VMEM((EB, D), jnp.float32),
        pltpu.VMEM((ZROWS, D), jnp.float32),
        pltpu.VMEM_SHARED((N_PAD, D), jnp.float32),
        pltpu.SemaphoreType.DMA,
        pltpu.SemaphoreType.DMA,
    ],
)(_sc_layer_body)


def _combine_body(last, p_ref, acc_ref, e_ref, acc_out_ref):
  e = p_ref[0] + p_ref[1]
  e_ref[...] = e
  a = acc_ref[...] + e
  if last:
    a = a * jnp.float32(1.0 / (N_LAYERS + 1))
  acc_out_ref[...] = a


def _combine(p, acc, last):
  rb = 1000
  grid = (N_NODES // rb,)
  return pl.pallas_call(
      functools.partial(_combine_body, last),
      grid=grid,
      in_specs=[
          pl.BlockSpec((NC, rb, D), lambda i: (0, i, 0)),
          pl.BlockSpec((rb, D), lambda i: (i, 0)),
      ],
      out_specs=[
          pl.BlockSpec((rb, D), lambda i: (i, 0)),
          pl.BlockSpec((rb, D), lambda i: (i, 0)),
      ],
      out_shape=[
          jax.ShapeDtypeStruct((N_NODES, D), jnp.float32),
          jax.ShapeDtypeStruct((N_NODES, D), jnp.float32),
      ],
  )(p, acc)


@jax.jit
def kernel(user_emb, item_emb, ingre_emb, edge_values, edge_index):
  all0 = jnp.concatenate([user_emb, item_emb, ingre_emb], axis=0)
  src = edge_index[0]
  dst = edge_index[1]
  emb = all0
  acc = all0
  for layer in range(N_LAYERS):
    p = _sc_layer(emb, src, dst, edge_values)
    emb, acc = _combine(p, acc, layer == N_LAYERS - 1)
  return (acc[:NUM_USERS],
          acc[NUM_USERS:NUM_USERS + NUM_ITEMS],
          acc[NUM_USERS + NUM_ITEMS:])


# speedup vs baseline: 1.3031x; 1.0016x over previous
import functools

import jax
import jax.numpy as jnp
from jax import lax
from jax.experimental import pallas as pl
from jax.experimental.pallas import tpu as pltpu
from jax.experimental.pallas import tpu_sc as plsc

NUM_USERS = 5000
NUM_ITEMS = 4000
NUM_INGRE = 1000
D = 128
N_LAYERS = 3
N_EDGES = 320000
N_NODES = NUM_USERS + NUM_ITEMS + NUM_INGRE

NC = 2
NS = 16
L = 16
NW = NC * NS
E_PER_W = N_EDGES // NW
EB = 80
N_BLOCKS = E_PER_W // EB
N_PAD = 10240
ROWS_PER_TILE = N_PAD // NS
ZROWS = 128


def _sc_layer_body(emb_hbm, src_hbm, dst_hbm, w_hbm, out_hbm,
                   src_v, dst_v, w_v, rows_v, zero_v, acc_sh, sem, sem_i):
  c = lax.axis_index("c")
  s = lax.axis_index("s")
  wid = s * NC + c

  def _zero_row(r, _):
    for k in range(D // L):
      zero_v[r, pl.ds(k * L, L)] = jnp.zeros((L,), jnp.float32)
    return 0
  lax.fori_loop(0, ZROWS, _zero_row, 0)
  for j in range(ROWS_PER_TILE // ZROWS):
    pltpu.sync_copy(zero_v, acc_sh.at[pl.ds(s * ROWS_PER_TILE + j * ZROWS,
                                            ZROWS)])
  plsc.subcore_barrier()

  def issue_idx(i):
    base = wid * E_PER_W + i * EB
    pltpu.async_copy(src_hbm.at[pl.ds(base, EB)], src_v, sem_i)
    pltpu.async_copy(dst_hbm.at[pl.ds(base, EB)], dst_v, sem_i)
    pltpu.async_copy(w_hbm.at[pl.ds(base, EB)], w_v, sem_i)

  issue_idx(0)
  def _block(i, _):
    pltpu.make_async_copy(src_hbm.at[pl.ds(0, EB)], src_v, sem_i).wait()
    pltpu.make_async_copy(src_hbm.at[pl.ds(0, EB)], dst_v, sem_i).wait()
    pltpu.make_async_copy(w_hbm.at[pl.ds(0, EB)], w_v, sem_i).wait()
    pltpu.async_copy(emb_hbm.at[src_v], rows_v, sem).wait()

    def _scale(g, _):
      w16 = w_v[pl.ds(g * L, L)]
      for j in range(L):
        wb = w16[j]
        b = g * L + j
        for k in range(D // L):
          rows_v[b, pl.ds(k * L, L)] = rows_v[b, pl.ds(k * L, L)] * wb
      return 0
    lax.fori_loop(0, EB // L, _scale, 0)

    pltpu.sync_copy(rows_v, acc_sh.at[dst_v], add=True)
    @pl.when(i + 1 < N_BLOCKS)
    def _():
      issue_idx(i + 1)
    return 0
  lax.fori_loop(0, N_BLOCKS, _block, 0)
  plsc.subcore_barrier()

  r0 = s * ROWS_PER_TILE
  @pl.when(s < NS - 1)
  def _():
    pltpu.sync_copy(acc_sh.at[pl.ds(r0, ROWS_PER_TILE)],
                    out_hbm.at[c, pl.ds(r0, ROWS_PER_TILE)])
  @pl.when(s == NS - 1)
  def _():
    pltpu.sync_copy(acc_sh.at[pl.ds(r0, N_NODES - (NS - 1) * ROWS_PER_TILE)],
                    out_hbm.at[c, pl.ds(r0, N_NODES - (NS - 1) * ROWS_PER_TILE)])


_sc_layer = functools.partial(
    pl.kernel,
    out_type=jax.ShapeDtypeStruct((NC, N_NODES, D), jnp.float32),
    mesh=plsc.VectorSubcoreMesh(core_axis_name="c", subcore_axis_name="s",
                                num_cores=NC, num_subcores=NS),
    scratch_types=[
        pltpu.VMEM((EB,), jnp.int32),
        pltpu.VMEM((EB,), jnp.int32),
        pltpu.VMEM((EB,), jnp.float32),
        pltpu.VMEM((EB, D), jnp.float32),
        pltpu.VMEM((ZROWS, D), jnp.float32),
        pltpu.VMEM_SHARED((N_PAD, D), jnp.float32),
        pltpu.SemaphoreType.DMA,
        pltpu.SemaphoreType.DMA,
    ],
)(_sc_layer_body)


def _combine_body(last, p_ref, acc_ref, e_ref, acc_out_ref):
  e = p_ref[0] + p_ref[1]
  e_ref[...] = e
  a = acc_ref[...] + e
  if last:
    a = a * jnp.float32(1.0 / (N_LAYERS + 1))
  acc_out_ref[...] = a


def _combine(p, acc, last):
  rb = 1000
  grid = (N_NODES // rb,)
  return pl.pallas_call(
      functools.partial(_combine_body, last),
      grid=grid,
      in_specs=[
          pl.BlockSpec((NC, rb, D), lambda i: (0, i, 0)),
          pl.BlockSpec((rb, D), lambda i: (i, 0)),
      ],
      out_specs=[
          pl.BlockSpec((rb, D), lambda i: (i, 0)),
          pl.BlockSpec((rb, D), lambda i: (i, 0)),
      ],
      out_shape=[
          jax.ShapeDtypeStruct((N_NODES, D), jnp.float32),
          jax.ShapeDtypeStruct((N_NODES, D), jnp.float32),
      ],
  )(p, acc)


@jax.jit
def kernel(user_emb, item_emb, ingre_emb, edge_values, edge_index):
  all0 = jnp.concatenate([user_emb, item_emb, ingre_emb], axis=0)
  src = edge_index[0]
  dst = edge_index[1]
  emb = all0
  acc = all0
  for layer in range(N_LAYERS):
    p = _sc_layer(emb, src, dst, edge_values)
    emb, acc = _combine(p, acc, layer == N_LAYERS - 1)
  return (acc[:NUM_USERS],
          acc[NUM_USERS:NUM_USERS + NUM_ITEMS],
          acc[NUM_USERS + NUM_ITEMS:])


# speedup vs baseline: 2.1410x; 1.6430x over previous
import functools

import jax
import jax.numpy as jnp
from jax import lax
from jax.experimental import pallas as pl
from jax.experimental.pallas import tpu as pltpu
from jax.experimental.pallas import tpu_sc as plsc

NUM_USERS = 5000
NUM_ITEMS = 4000
NUM_INGRE = 1000
D = 128
N_LAYERS = 3
N_EDGES = 320000
N_NODES = NUM_USERS + NUM_ITEMS + NUM_INGRE

NC = 2
NS = 16
L = 16
NW = NC * NS
E_PER_W = N_EDGES // NW
EB = 80
N_BLOCKS = E_PER_W // EB
N_PAD = 10240
ROWS_PER_TILE = N_PAD // NS
ZROWS = 128


def _sc_layer_body(emb_hbm, src_hbm, dst_hbm, w_hbm, out_hbm,
                   src_a, src_b, dst_v, w_a, w_b, rows_a, rows_b, zero_v,
                   acc_sh, sem_g, sem_i, sem_d):
  c = lax.axis_index("c")
  s = lax.axis_index("s")
  wid = s * NC + c

  def _zero_row(r, _):
    for k in range(D // L):
      zero_v[r, pl.ds(k * L, L)] = jnp.zeros((L,), jnp.float32)
    return 0
  lax.fori_loop(0, ZROWS, _zero_row, 0)
  for j in range(ROWS_PER_TILE // ZROWS):
    pltpu.sync_copy(zero_v, acc_sh.at[pl.ds(s * ROWS_PER_TILE + j * ZROWS,
                                            ZROWS)])
  plsc.subcore_barrier()

  srcs = (src_a, src_b)
  ws = (w_a, w_b)
  rows = (rows_a, rows_b)

  def issue_sw(i, b):
    base = wid * E_PER_W + i * EB
    pltpu.async_copy(src_hbm.at[pl.ds(base, EB)], srcs[b], sem_i)
    pltpu.async_copy(w_hbm.at[pl.ds(base, EB)], ws[b], sem_i)

  def wait_sw(b):
    pltpu.make_async_copy(src_hbm.at[pl.ds(0, EB)], srcs[b], sem_i).wait()
    pltpu.make_async_copy(w_hbm.at[pl.ds(0, EB)], ws[b], sem_i).wait()

  def issue_dst(i):
    base = wid * E_PER_W + i * EB
    pltpu.async_copy(dst_hbm.at[pl.ds(base, EB)], dst_v, sem_d)

  def scale(b):
    wv = ws[b]
    rv = rows[b]
    def _g(g, _):
      w16 = wv[pl.ds(g * L, L)]
      for j in range(L):
        wb = w16[j]
        r = g * L + j
        for k in range(D // L):
          rv[r, pl.ds(k * L, L)] = rv[r, pl.ds(k * L, L)] * wb
      return 0
    lax.fori_loop(0, EB // L, _g, 0)

  issue_sw(0, 0)
  wait_sw(0)
  pltpu.async_copy(emb_hbm.at[src_a], rows_a, sem_g)
  issue_sw(1, 1)
  issue_dst(0)

  def _pair(jp, _):
    for b in range(2):
      i = 2 * jp + b
      pltpu.make_async_copy(emb_hbm.at[pl.ds(0, EB)], rows[b], sem_g).wait()
      @pl.when(i + 1 < N_BLOCKS)
      def _():
        wait_sw(b ^ 1)
        pltpu.async_copy(emb_hbm.at[srcs[b ^ 1]], rows[b ^ 1], sem_g)
      scale(b)
      pltpu.make_async_copy(src_hbm.at[pl.ds(0, EB)], dst_v, sem_d).wait()
      pltpu.sync_copy(rows[b], acc_sh.at[dst_v], add=True)
      @pl.when(i + 1 < N_BLOCKS)
      def _():
        issue_dst(i + 1)
      @pl.when(i + 2 < N_BLOCKS)
      def _():
        issue_sw(i + 2, b)
    return 0
  lax.fori_loop(0, N_BLOCKS // 2, _pair, 0)
  pltpu.make_async_copy(emb_hbm.at[pl.ds(0, EB)], rows[0], sem_g).wait()
  scale(0)
  pltpu.make_async_copy(src_hbm.at[pl.ds(0, EB)], dst_v, sem_d).wait()
  pltpu.sync_copy(rows[0], acc_sh.at[dst_v], add=True)
  plsc.subcore_barrier()

  r0 = s * ROWS_PER_TILE
  @pl.when(s < NS - 1)
  def _():
    pltpu.sync_copy(acc_sh.at[pl.ds(r0, ROWS_PER_TILE)],
                    out_hbm.at[c, pl.ds(r0, ROWS_PER_TILE)])
  @pl.when(s == NS - 1)
  def _():
    pltpu.sync_copy(acc_sh.at[pl.ds(r0, N_NODES - (NS - 1) * ROWS_PER_TILE)],
                    out_hbm.at[c, pl.ds(r0, N_NODES - (NS - 1) * ROWS_PER_TILE)])


_sc_layer = functools.partial(
    pl.kernel,
    out_type=jax.ShapeDtypeStruct((NC, N_NODES, D), jnp.float32),
    mesh=plsc.VectorSubcoreMesh(core_axis_name="c", subcore_axis_name="s",
                                num_cores=NC, num_subcores=NS),
    scratch_types=[
        pltpu.VMEM((EB,), jnp.int32),
        pltpu.VMEM((EB,), jnp.int32),
        pltpu.VMEM((EB,), jnp.int32),
        pltpu.VMEM((EB,), jnp.float32),
        pltpu.VMEM((EB,), jnp.float32),
        pltpu.VMEM((EB, D), jnp.float32),
        pltpu.VMEM((EB, D), jnp.float32),
        pltpu.VMEM((ZROWS, D), jnp.float32),
        pltpu.VMEM_SHARED((N_PAD, D), jnp.float32),
        pltpu.SemaphoreType.DMA,
        pltpu.SemaphoreType.DMA,
        pltpu.SemaphoreType.DMA,
    ],
)(_sc_layer_body)


def _combine_body(last, p_ref, acc_ref, e_ref, acc_out_ref):
  e = p_ref[0] + p_ref[1]
  e_ref[...] = e
  a = acc_ref[...] + e
  if last:
    a = a * jnp.float32(1.0 / (N_LAYERS + 1))
  acc_out_ref[...] = a


def _combine(p, acc, last):
  rb = 1000
  grid = (N_NODES // rb,)
  return pl.pallas_call(
      functools.partial(_combine_body, last),
      grid=grid,
      in_specs=[
          pl.BlockSpec((NC, rb, D), lambda i: (0, i, 0)),
          pl.BlockSpec((rb, D), lambda i: (i, 0)),
      ],
      out_specs=[
          pl.BlockSpec((rb, D), lambda i: (i, 0)),
          pl.BlockSpec((rb, D), lambda i: (i, 0)),
      ],
      out_shape=[
          jax.ShapeDtypeStruct((N_NODES, D), jnp.float32),
          jax.ShapeDtypeStruct((N_NODES, D), jnp.float32),
      ],
  )(p, acc)


@jax.jit
def kernel(user_emb, item_emb, ingre_emb, edge_values, edge_index):
  all0 = jnp.concatenate([user_emb, item_emb, ingre_emb], axis=0)
  src = edge_index[0]
  dst = edge_index[1]
  emb = all0
  acc = all0
  for layer in range(N_LAYERS):
    p = _sc_layer(emb, src, dst, edge_values)
    emb, acc = _combine(p, acc, layer == N_LAYERS - 1)
  return (acc[:NUM_USERS],
          acc[NUM_USERS:NUM_USERS + NUM_ITEMS],
          acc[NUM_USERS + NUM_ITEMS:])


# speedup vs baseline: 2.1415x; 1.0002x over previous
import functools

import jax
import jax.numpy as jnp
from jax import lax
from jax.experimental import pallas as pl
from jax.experimental.pallas import tpu as pltpu
from jax.experimental.pallas import tpu_sc as plsc

NUM_USERS = 5000
NUM_ITEMS = 4000
NUM_INGRE = 1000
D = 128
N_LAYERS = 3
N_EDGES = 320000
N_NODES = NUM_USERS + NUM_ITEMS + NUM_INGRE

NC = 2
NS = 16
L = 16
NW = NC * NS
E_PER_W = N_EDGES // NW
EB = 80
N_BLOCKS = E_PER_W // EB
N_PAD = 10240
ROWS_PER_TILE = N_PAD // NS
ZROWS = 128


def _sc_layer_body(emb_hbm, src_hbm, dst_hbm, w_hbm, out_hbm,
                   src_a, src_b, src_c, dst_v, w_a, w_b, w_c,
                   rows_a, rows_b, rows_c, zero_v,
                   acc_sh, sem_g, sem_i, sem_d):
  c = lax.axis_index("c")
  s = lax.axis_index("s")
  wid = s * NC + c

  def _zero_row(r, _):
    for k in range(D // L):
      zero_v[r, pl.ds(k * L, L)] = jnp.zeros((L,), jnp.float32)
    return 0
  lax.fori_loop(0, ZROWS, _zero_row, 0)
  for j in range(ROWS_PER_TILE // ZROWS):
    pltpu.sync_copy(zero_v, acc_sh.at[pl.ds(s * ROWS_PER_TILE + j * ZROWS,
                                            ZROWS)])
  plsc.subcore_barrier()

  srcs = (src_a, src_b, src_c)
  ws = (w_a, w_b, w_c)
  rows = (rows_a, rows_b, rows_c)

  def issue_sw(i, b):
    base = wid * E_PER_W + i * EB
    pltpu.async_copy(src_hbm.at[pl.ds(base, EB)], srcs[b], sem_i)
    pltpu.async_copy(w_hbm.at[pl.ds(base, EB)], ws[b], sem_i)

  def wait_sw(b):
    pltpu.make_async_copy(src_hbm.at[pl.ds(0, EB)], srcs[b], sem_i).wait()
    pltpu.make_async_copy(w_hbm.at[pl.ds(0, EB)], ws[b], sem_i).wait()

  def issue_dst(i):
    base = wid * E_PER_W + i * EB
    pltpu.async_copy(dst_hbm.at[pl.ds(base, EB)], dst_v, sem_d)

  def scale(b):
    wv = ws[b]
    rv = rows[b]
    def _g(g, _):
      w16 = wv[pl.ds(g * L, L)]
      for j in range(L):
        wb = w16[j]
        r = g * L + j
        for k in range(D // L):
          rv[r, pl.ds(k * L, L)] = rv[r, pl.ds(k * L, L)] * wb
      return 0
    lax.fori_loop(0, EB // L, _g, 0)

  issue_sw(0, 0)
  issue_sw(1, 1)
  issue_sw(2, 2)
  wait_sw(0)
  pltpu.async_copy(emb_hbm.at[src_a], rows_a, sem_g)
  wait_sw(1)
  pltpu.async_copy(emb_hbm.at[src_b], rows_b, sem_g)
  issue_dst(0)

  def body(i, r):
    pltpu.make_async_copy(emb_hbm.at[pl.ds(0, EB)], rows[r], sem_g).wait()
    r2 = (r + 2) % 3
    @pl.when(i + 2 < N_BLOCKS)
    def _():
      wait_sw(r2)
      pltpu.async_copy(emb_hbm.at[srcs[r2]], rows[r2], sem_g)
    scale(r)
    pltpu.make_async_copy(src_hbm.at[pl.ds(0, EB)], dst_v, sem_d).wait()
    pltpu.sync_copy(rows[r], acc_sh.at[dst_v], add=True)
    @pl.when(i + 1 < N_BLOCKS)
    def _():
      issue_dst(i + 1)
    @pl.when(i + 3 < N_BLOCKS)
    def _():
      issue_sw(i + 3, r)

  def _trip(jp, _):
    for r in range(3):
      body(3 * jp + r, r)
    return 0
  lax.fori_loop(0, N_BLOCKS // 3, _trip, 0)
  body(N_BLOCKS - 2, 0)
  body(N_BLOCKS - 1, 1)
  plsc.subcore_barrier()

  r0 = s * ROWS_PER_TILE
  @pl.when(s < NS - 1)
  def _():
    pltpu.sync_copy(acc_sh.at[pl.ds(r0, ROWS_PER_TILE)],
                    out_hbm.at[c, pl.ds(r0, ROWS_PER_TILE)])
  @pl.when(s == NS - 1)
  def _():
    pltpu.sync_copy(acc_sh.at[pl.ds(r0, N_NODES - (NS - 1) * ROWS_PER_TILE)],
                    out_hbm.at[c, pl.ds(r0, N_NODES - (NS - 1) * ROWS_PER_TILE)])


_sc_layer = functools.partial(
    pl.kernel,
    out_type=jax.ShapeDtypeStruct((NC, N_NODES, D), jnp.float32),
    mesh=plsc.VectorSubcoreMesh(core_axis_name="c", subcore_axis_name="s",
                                num_cores=NC, num_subcores=NS),
    scratch_types=[
        pltpu.VMEM((EB,), jnp.int32),
        pltpu.VMEM((EB,), jnp.int32),
        pltpu.VMEM((EB,), jnp.int32),
        pltpu.VMEM((EB,), jnp.int32),
        pltpu.VMEM((EB,), jnp.float32),
        pltpu.VMEM((EB,), jnp.float32),
        pltpu.VMEM((EB,), jnp.float32),
        pltpu.VMEM((EB, D), jnp.float32),
        pltpu.VMEM((EB, D), jnp.float32),
        pltpu.VMEM((EB, D), jnp.float32),
        pltpu.VMEM((ZROWS, D), jnp.float32),
        pltpu.VMEM_SHARED((N_PAD, D), jnp.float32),
        pltpu.SemaphoreType.DMA,
        pltpu.SemaphoreType.DMA,
        pltpu.SemaphoreType.DMA,
    ],
)(_sc_layer_body)


def _combine_body(last, p_ref, acc_ref, e_ref, acc_out_ref):
  e = p_ref[0] + p_ref[1]
  e_ref[...] = e
  a = acc_ref[...] + e
  if last:
    a = a * jnp.float32(1.0 / (N_LAYERS + 1))
  acc_out_ref[...] = a


def _combine(p, acc, last):
  rb = 1000
  grid = (N_NODES // rb,)
  return pl.pallas_call(
      functools.partial(_combine_body, last),
      grid=grid,
      in_specs=[
          pl.BlockSpec((NC, rb, D), lambda i: (0, i, 0)),
          pl.BlockSpec((rb, D), lambda i: (i, 0)),
      ],
      out_specs=[
          pl.BlockSpec((rb, D), lambda i: (i, 0)),
          pl.BlockSpec((rb, D), lambda i: (i, 0)),
      ],
      out_shape=[
          jax.ShapeDtypeStruct((N_NODES, D), jnp.float32),
          jax.ShapeDtypeStruct((N_NODES, D), jnp.float32),
      ],
  )(p, acc)


@jax.jit
def kernel(user_emb, item_emb, ingre_emb, edge_values, edge_index):
  all0 = jnp.concatenate([user_emb, item_emb, ingre_emb], axis=0)
  src = edge_index[0]
  dst = edge_index[1]
  emb = all0
  acc = all0
  for layer in range(N_LAYERS):
    p = _sc_layer(emb, src, dst, edge_values)
    emb, acc = _combine(p, acc, layer == N_LAYERS - 1)
  return (acc[:NUM_USERS],
          acc[NUM_USERS:NUM_USERS + NUM_ITEMS],
          acc[NUM_USERS + NUM_ITEMS:])
